# baseline (device time: 30513 ns/iter reference)
import jax
import jax.numpy as jnp
from jax import lax
from jax.experimental import pallas as pl
from jax.experimental.pallas import tpu as pltpu

X_SIZE = 2


def kernel(x, router, W1, W2):
    t_per, d = x.shape
    e_per, _, f = W1.shape
    t_total = t_per * X_SIZE

    def body(x_ref, router_ref, w1_ref, w2_ref, out_ref,
             xo_ref, ro_ref, po_ref, acc_ref, send_sems, recv_sems):
        my_x = lax.axis_index("x")
        my_y = lax.axis_index("y")
        my_z = lax.axis_index("z")
        partner = (1 - my_x, my_y, my_z)

        barrier_sem = pltpu.get_barrier_semaphore()
        pl.semaphore_signal(barrier_sem, inc=1, device_id=partner,
                            device_id_type=pl.DeviceIdType.MESH)
        pl.semaphore_wait(barrier_sem, 1)

        r_router = pltpu.make_async_remote_copy(
            src_ref=router_ref, dst_ref=ro_ref,
            send_sem=send_sems.at[0], recv_sem=recv_sems.at[0],
            device_id=partner, device_id_type=pl.DeviceIdType.MESH)
        r_router.start()
        r_tok = pltpu.make_async_remote_copy(
            src_ref=x_ref, dst_ref=xo_ref,
            send_sem=send_sems.at[1], recv_sem=recv_sems.at[1],
            device_id=partner, device_id_type=pl.DeviceIdType.MESH)
        r_tok.start()
        r_router.wait()
        r_tok.wait()

        is0 = my_x == 0
        x_mine = x_ref[...]
        x_oth = xo_ref[...]
        x_all = jnp.where(is0,
                          jnp.concatenate([x_mine, x_oth], axis=0),
                          jnp.concatenate([x_oth, x_mine], axis=0))
        r_mine = router_ref[...]
        r_oth = ro_ref[...]
        r_full = jnp.where(is0,
                           jnp.concatenate([r_mine, r_oth], axis=1),
                           jnp.concatenate([r_oth, r_mine], axis=1))

        gates = jnp.dot(x_all, r_full, preferred_element_type=jnp.float32)
        i1 = jnp.argmax(gates, axis=1)
        m1 = jnp.max(gates, axis=1)
        cols = lax.broadcasted_iota(jnp.int32, gates.shape, 1)
        masked = jnp.where(cols == i1[:, None], -jnp.inf, gates)
        i2 = jnp.argmax(masked, axis=1)
        m2 = jnp.max(masked, axis=1)
        e2 = jnp.exp(m2 - m1)
        w_top = 1.0 / (1.0 + e2)
        w_snd = e2 / (1.0 + e2)

        acc = jnp.zeros((t_total, d), jnp.float32)
        for j in range(e_per):
            e_glob = my_x * e_per + j
            w_e = (jnp.where(i1 == e_glob, w_top, 0.0)
                   + jnp.where(i2 == e_glob, w_snd, 0.0))
            h = jnp.maximum(
                jnp.dot(x_all, w1_ref[j], preferred_element_type=jnp.float32),
                0.0)
            acc = acc + jnp.dot(
                h, w2_ref[j], preferred_element_type=jnp.float32) * w_e[:, None]
        acc_ref[...] = acc

        other_start = (1 - my_x) * t_per
        r_par = pltpu.make_async_remote_copy(
            src_ref=acc_ref.at[pl.ds(other_start, t_per), :],
            dst_ref=po_ref,
            send_sem=send_sems.at[2], recv_sem=recv_sems.at[2],
            device_id=partner, device_id_type=pl.DeviceIdType.MESH)
        r_par.start()
        r_par.wait()

        out_ref[...] = acc_ref[pl.ds(my_x * t_per, t_per), :] + po_ref[...]

    return pl.pallas_call(
        body,
        out_shape=jax.ShapeDtypeStruct((t_per, d), jnp.float32),
        in_specs=[pl.BlockSpec(memory_space=pltpu.VMEM)] * 4,
        out_specs=pl.BlockSpec(memory_space=pltpu.VMEM),
        scratch_shapes=[
            pltpu.VMEM((t_per, d), jnp.float32),
            pltpu.VMEM((d, e_per), jnp.float32),
            pltpu.VMEM((t_per, d), jnp.float32),
            pltpu.VMEM((t_total, d), jnp.float32),
            pltpu.SemaphoreType.DMA((3,)),
            pltpu.SemaphoreType.DMA((3,)),
        ],
        compiler_params=pltpu.CompilerParams(collective_id=0),
    )(x, router, W1, W2)


# device time: 26152 ns/iter; 1.1668x vs baseline; 1.1668x over previous
import jax
import jax.numpy as jnp
from jax import lax
from jax.experimental import pallas as pl
from jax.experimental.pallas import tpu as pltpu

X_SIZE = 2


def kernel(x, router, W1, W2):
    t_per, d = x.shape
    e_per, _, f = W1.shape
    n_exp = e_per * X_SIZE

    def body(x_ref, router_ref, w1_ref, w2_ref, out_ref,
             ro_ref, xb_ref, xo_ref, gm_ref, go_ref, pb_ref, po_ref,
             send_sems, recv_sems):
        my_x = lax.axis_index("x")
        my_y = lax.axis_index("y")
        my_z = lax.axis_index("z")
        partner = (1 - my_x, my_y, my_z)

        barrier_sem = pltpu.get_barrier_semaphore()
        pl.semaphore_signal(barrier_sem, inc=1, device_id=partner,
                            device_id_type=pl.DeviceIdType.MESH)
        pl.semaphore_wait(barrier_sem, 1)

        r_router = pltpu.make_async_remote_copy(
            src_ref=router_ref, dst_ref=ro_ref,
            send_sem=send_sems.at[0], recv_sem=recv_sems.at[0],
            device_id=partner, device_id_type=pl.DeviceIdType.MESH)
        r_router.start()

        x_mine = x_ref[...]
        xb_ref[...] = x_mine.astype(jnp.bfloat16)
        r_router.wait()

        is0 = my_x == 0
        r_mine = router_ref[...]
        r_oth = ro_ref[...]
        r_full = jnp.where(is0,
                           jnp.concatenate([r_mine, r_oth], axis=1),
                           jnp.concatenate([r_oth, r_mine], axis=1))
        gm = jnp.dot(x_mine, r_full, preferred_element_type=jnp.float32)
        gm_ref[...] = gm

        r_tok = pltpu.make_async_remote_copy(
            src_ref=xb_ref, dst_ref=xo_ref,
            send_sem=send_sems.at[1], recv_sem=recv_sems.at[1],
            device_id=partner, device_id_type=pl.DeviceIdType.MESH)
        r_tok.start()
        r_gate = pltpu.make_async_remote_copy(
            src_ref=gm_ref, dst_ref=go_ref,
            send_sem=send_sems.at[2], recv_sem=recv_sems.at[2],
            device_id=partner, device_id_type=pl.DeviceIdType.MESH)
        r_gate.start()

        w1b = w1_ref[...].astype(jnp.bfloat16)
        w2b = w2_ref[...].astype(jnp.bfloat16)

        def expert_weights(gates):
            i1 = jnp.argmax(gates, axis=1)
            m1 = jnp.max(gates, axis=1)
            cols = lax.broadcasted_iota(jnp.int32, gates.shape, 1)
            masked = jnp.where(cols == i1[:, None], -jnp.inf, gates)
            i2 = jnp.argmax(masked, axis=1)
            m2 = jnp.max(masked, axis=1)
            e2 = jnp.exp(m2 - m1)
            w_top = 1.0 / (1.0 + e2)
            w_snd = e2 / (1.0 + e2)
            return i1, i2, w_top, w_snd

        def local_experts(tok_b, gates):
            i1, i2, w_top, w_snd = expert_weights(gates)
            acc = jnp.zeros((tok_b.shape[0], d), jnp.float32)
            for j in range(e_per):
                e_glob = my_x * e_per + j
                w_e = (jnp.where(i1 == e_glob, w_top, 0.0)
                       + jnp.where(i2 == e_glob, w_snd, 0.0))
                h = jnp.maximum(
                    jnp.dot(tok_b, w1b[j], preferred_element_type=jnp.float32),
                    0.0).astype(jnp.bfloat16)
                acc = acc + jnp.dot(
                    h, w2b[j],
                    preferred_element_type=jnp.float32) * w_e[:, None]
            return acc

        acc_my = local_experts(xb_ref[...], gm)

        r_tok.wait()
        r_gate.wait()
        acc_for = local_experts(xo_ref[...], go_ref[...])
        pb_ref[...] = acc_for.astype(jnp.bfloat16)
        r_par = pltpu.make_async_remote_copy(
            src_ref=pb_ref, dst_ref=po_ref,
            send_sem=send_sems.at[3], recv_sem=recv_sems.at[3],
            device_id=partner, device_id_type=pl.DeviceIdType.MESH)
        r_par.start()
        r_par.wait()

        out_ref[...] = acc_my + po_ref[...].astype(jnp.float32)

    return pl.pallas_call(
        body,
        out_shape=jax.ShapeDtypeStruct((t_per, d), jnp.float32),
        in_specs=[pl.BlockSpec(memory_space=pltpu.VMEM)] * 4,
        out_specs=pl.BlockSpec(memory_space=pltpu.VMEM),
        scratch_shapes=[
            pltpu.VMEM((d, e_per), jnp.float32),
            pltpu.VMEM((t_per, d), jnp.bfloat16),
            pltpu.VMEM((t_per, d), jnp.bfloat16),
            pltpu.VMEM((t_per, n_exp), jnp.float32),
            pltpu.VMEM((t_per, n_exp), jnp.float32),
            pltpu.VMEM((t_per, d), jnp.bfloat16),
            pltpu.VMEM((t_per, d), jnp.bfloat16),
            pltpu.SemaphoreType.DMA((4,)),
            pltpu.SemaphoreType.DMA((4,)),
        ],
        compiler_params=pltpu.CompilerParams(collective_id=0),
    )(x, router, W1, W2)


# device time: 24855 ns/iter; 1.2276x vs baseline; 1.0522x over previous
import jax
import jax.numpy as jnp
from jax import lax
from jax.experimental import pallas as pl
from jax.experimental.pallas import tpu as pltpu

X_SIZE = 2


def kernel(x, router, W1, W2):
    t_per, d = x.shape
    e_per, _, f = W1.shape
    n_exp = e_per * X_SIZE

    def body(x_ref, router_ref, w1_ref, w2_ref, out_ref,
             ro_ref, xb_ref, xo_ref, gm_ref, go_ref, pb_ref, po_ref,
             send_sems, recv_sems):
        my_x = lax.axis_index("x")
        my_y = lax.axis_index("y")
        my_z = lax.axis_index("z")
        partner = (1 - my_x, my_y, my_z)

        barrier_sem = pltpu.get_barrier_semaphore()
        pl.semaphore_signal(barrier_sem, inc=1, device_id=partner,
                            device_id_type=pl.DeviceIdType.MESH)

        x_mine = x_ref[...]
        xb = x_mine.astype(jnp.bfloat16)
        xb_ref[...] = xb
        w1b = w1_ref[...].astype(jnp.bfloat16)
        w2b = w2_ref[...].astype(jnp.bfloat16)

        def expert_outputs(tok_b):
            ys = []
            for j in range(e_per):
                h = jnp.maximum(
                    jnp.dot(tok_b, w1b[j], preferred_element_type=jnp.float32),
                    0.0).astype(jnp.bfloat16)
                ys.append(jnp.dot(h, w2b[j],
                                  preferred_element_type=jnp.float32))
            return ys

        def expert_weights(gates):
            i1 = jnp.argmax(gates, axis=1)
            m1 = jnp.max(gates, axis=1)
            cols = lax.broadcasted_iota(jnp.int32, gates.shape, 1)
            masked = jnp.where(cols == i1[:, None], -jnp.inf, gates)
            i2 = jnp.argmax(masked, axis=1)
            m2 = jnp.max(masked, axis=1)
            e2 = jnp.exp(m2 - m1)
            w_top = 1.0 / (1.0 + e2)
            w_snd = e2 / (1.0 + e2)
            ws = []
            for j in range(e_per):
                e_glob = my_x * e_per + j
                ws.append(jnp.where(i1 == e_glob, w_top, 0.0)
                          + jnp.where(i2 == e_glob, w_snd, 0.0))
            return ws

        y_mine = expert_outputs(xb)

        pl.semaphore_wait(barrier_sem, 1)

        r_router = pltpu.make_async_remote_copy(
            src_ref=router_ref, dst_ref=ro_ref,
            send_sem=send_sems.at[0], recv_sem=recv_sems.at[0],
            device_id=partner, device_id_type=pl.DeviceIdType.MESH)
        r_router.start()
        r_tok = pltpu.make_async_remote_copy(
            src_ref=xb_ref, dst_ref=xo_ref,
            send_sem=send_sems.at[1], recv_sem=recv_sems.at[1],
            device_id=partner, device_id_type=pl.DeviceIdType.MESH)
        r_tok.start()

        r_router.wait()
        is0 = my_x == 0
        r_mine = router_ref[...]
        r_oth = ro_ref[...]
        r_full = jnp.where(is0,
                           jnp.concatenate([r_mine, r_oth], axis=1),
                           jnp.concatenate([r_oth, r_mine], axis=1))
        gm = jnp.dot(x_mine, r_full, preferred_element_type=jnp.float32)
        gm_ref[...] = gm
        r_gate = pltpu.make_async_remote_copy(
            src_ref=gm_ref, dst_ref=go_ref,
            send_sem=send_sems.at[2], recv_sem=recv_sems.at[2],
            device_id=partner, device_id_type=pl.DeviceIdType.MESH)
        r_gate.start()

        r_tok.wait()
        r_gate.wait()
        w_for = expert_weights(go_ref[...])
        y_for = expert_outputs(xo_ref[...])
        acc_for = (y_for[0] * w_for[0][:, None]
                   + y_for[1] * w_for[1][:, None])
        pb_ref[...] = acc_for.astype(jnp.bfloat16)
        r_par = pltpu.make_async_remote_copy(
            src_ref=pb_ref, dst_ref=po_ref,
            send_sem=send_sems.at[3], recv_sem=recv_sems.at[3],
            device_id=partner, device_id_type=pl.DeviceIdType.MESH)
        r_par.start()

        w_mine = expert_weights(gm)
        acc_my = (y_mine[0] * w_mine[0][:, None]
                  + y_mine[1] * w_mine[1][:, None])

        r_par.wait()
        out_ref[...] = acc_my + po_ref[...].astype(jnp.float32)

    return pl.pallas_call(
        body,
        out_shape=jax.ShapeDtypeStruct((t_per, d), jnp.float32),
        in_specs=[pl.BlockSpec(memory_space=pltpu.VMEM)] * 4,
        out_specs=pl.BlockSpec(memory_space=pltpu.VMEM),
        scratch_shapes=[
            pltpu.VMEM((d, e_per), jnp.float32),
            pltpu.VMEM((t_per, d), jnp.bfloat16),
            pltpu.VMEM((t_per, d), jnp.bfloat16),
            pltpu.VMEM((t_per, n_exp), jnp.float32),
            pltpu.VMEM((t_per, n_exp), jnp.float32),
            pltpu.VMEM((t_per, d), jnp.bfloat16),
            pltpu.VMEM((t_per, d), jnp.bfloat16),
            pltpu.SemaphoreType.DMA((4,)),
            pltpu.SemaphoreType.DMA((4,)),
        ],
        compiler_params=pltpu.CompilerParams(collective_id=0),
    )(x, router, W1, W2)


# device time: 24354 ns/iter; 1.2529x vs baseline; 1.0206x over previous
import jax
import jax.numpy as jnp
from jax import lax
from jax.experimental import pallas as pl
from jax.experimental.pallas import tpu as pltpu

X_SIZE = 2


def kernel(x, router, W1, W2):
    t_per, d = x.shape
    e_per, _, f = W1.shape
    n_exp = e_per * X_SIZE
    half = t_per // 2

    def body(x_ref, router_ref, w1_ref, w2_ref, out_ref,
             ro_ref, xb_ref, xo_ref, gm_ref, go_ref, pb_ref, po_ref,
             send_sems, recv_sems):
        my_x = lax.axis_index("x")
        my_y = lax.axis_index("y")
        my_z = lax.axis_index("z")
        partner = (1 - my_x, my_y, my_z)

        def rdma(src, dst, i):
            return pltpu.make_async_remote_copy(
                src_ref=src, dst_ref=dst,
                send_sem=send_sems.at[i], recv_sem=recv_sems.at[i],
                device_id=partner, device_id_type=pl.DeviceIdType.MESH)

        barrier_sem = pltpu.get_barrier_semaphore()
        pl.semaphore_signal(barrier_sem, inc=1, device_id=partner,
                            device_id_type=pl.DeviceIdType.MESH)

        x_mine = x_ref[...]
        xb = x_mine.astype(jnp.bfloat16)
        xb_ref[...] = xb
        w1b = w1_ref[...].astype(jnp.bfloat16)
        w2b = w2_ref[...].astype(jnp.bfloat16)

        pl.semaphore_wait(barrier_sem, 1)

        r_router = rdma(router_ref, ro_ref, 0)
        r_router.start()
        r_tok1 = rdma(xb_ref.at[pl.ds(0, half), :],
                      xo_ref.at[pl.ds(0, half), :], 1)
        r_tok1.start()

        r_router.wait()
        is0 = my_x == 0
        r_mine = router_ref[...]
        r_oth = ro_ref[...]
        r_full = jnp.where(is0,
                           jnp.concatenate([r_mine, r_oth], axis=1),
                           jnp.concatenate([r_oth, r_mine], axis=1))
        gm = jnp.dot(x_mine, r_full, preferred_element_type=jnp.float32)
        gm_ref[...] = gm
        r_gate = rdma(gm_ref, go_ref, 2)
        r_gate.start()
        r_tok2 = rdma(xb_ref.at[pl.ds(half, half), :],
                      xo_ref.at[pl.ds(half, half), :], 3)
        r_tok2.start()

        def expert_outputs(tok_b):
            ys = []
            for j in range(e_per):
                h = jnp.maximum(
                    jnp.dot(tok_b, w1b[j], preferred_element_type=jnp.float32),
                    0.0).astype(jnp.bfloat16)
                ys.append(jnp.dot(h, w2b[j],
                                  preferred_element_type=jnp.float32))
            return ys

        def expert_weights(gates):
            i1 = jnp.argmax(gates, axis=1)
            m1 = jnp.max(gates, axis=1)
            cols = lax.broadcasted_iota(jnp.int32, gates.shape, 1)
            masked = jnp.where(cols == i1[:, None], -jnp.inf, gates)
            i2 = jnp.argmax(masked, axis=1)
            m2 = jnp.max(masked, axis=1)
            e2 = jnp.exp(m2 - m1)
            w_top = 1.0 / (1.0 + e2)
            w_snd = e2 / (1.0 + e2)
            ws = []
            for j in range(e_per):
                e_glob = my_x * e_per + j
                ws.append(jnp.where(i1 == e_glob, w_top, 0.0)
                          + jnp.where(i2 == e_glob, w_snd, 0.0))
            return ws

        y_my1 = expert_outputs(xb[:half])

        r_tok1.wait()
        r_gate.wait()
        w_for = expert_weights(go_ref[...])
        y_f1 = expert_outputs(xo_ref[pl.ds(0, half), :])
        acc_f1 = (y_f1[0] * w_for[0][:half, None]
                  + y_f1[1] * w_for[1][:half, None])
        pb_ref[pl.ds(0, half), :] = acc_f1.astype(jnp.bfloat16)
        r_par1 = rdma(pb_ref.at[pl.ds(0, half), :],
                      po_ref.at[pl.ds(0, half), :], 4)
        r_par1.start()

        r_tok2.wait()
        y_f2 = expert_outputs(xo_ref[pl.ds(half, half), :])
        acc_f2 = (y_f2[0] * w_for[0][half:, None]
                  + y_f2[1] * w_for[1][half:, None])
        pb_ref[pl.ds(half, half), :] = acc_f2.astype(jnp.bfloat16)
        r_par2 = rdma(pb_ref.at[pl.ds(half, half), :],
                      po_ref.at[pl.ds(half, half), :], 5)
        r_par2.start()

        y_my2 = expert_outputs(xb[half:])
        w_mine = expert_weights(gm)
        acc_my = jnp.concatenate(
            [y_my1[0] * w_mine[0][:half, None]
             + y_my1[1] * w_mine[1][:half, None],
             y_my2[0] * w_mine[0][half:, None]
             + y_my2[1] * w_mine[1][half:, None]], axis=0)

        r_par1.wait()
        r_par2.wait()
        out_ref[...] = acc_my + po_ref[...].astype(jnp.float32)

    return pl.pallas_call(
        body,
        out_shape=jax.ShapeDtypeStruct((t_per, d), jnp.float32),
        in_specs=[pl.BlockSpec(memory_space=pltpu.VMEM)] * 4,
        out_specs=pl.BlockSpec(memory_space=pltpu.VMEM),
        scratch_shapes=[
            pltpu.VMEM((d, e_per), jnp.float32),
            pltpu.VMEM((t_per, d), jnp.bfloat16),
            pltpu.VMEM((t_per, d), jnp.bfloat16),
            pltpu.VMEM((t_per, n_exp), jnp.float32),
            pltpu.VMEM((t_per, n_exp), jnp.float32),
            pltpu.VMEM((t_per, d), jnp.bfloat16),
            pltpu.VMEM((t_per, d), jnp.bfloat16),
            pltpu.SemaphoreType.DMA((6,)),
            pltpu.SemaphoreType.DMA((6,)),
        ],
        compiler_params=pltpu.CompilerParams(collective_id=0),
    )(x, router, W1, W2)


# device time: 24306 ns/iter; 1.2554x vs baseline; 1.0020x over previous
import jax
import jax.numpy as jnp
from jax import lax
from jax.experimental import pallas as pl
from jax.experimental.pallas import tpu as pltpu

X_SIZE = 2


def kernel(x, router, W1, W2):
    t_per, d = x.shape
    e_per, _, f = W1.shape
    n_exp = e_per * X_SIZE
    half = t_per // 2

    def body(x_ref, router_ref, w1_ref, w2_ref, out_ref,
             ro_ref, xb_ref, xo_ref, gm_ref, go_ref, pb_ref, po_ref,
             send_sems, recv_sems):
        my_x = lax.axis_index("x")
        my_y = lax.axis_index("y")
        my_z = lax.axis_index("z")
        partner = (1 - my_x, my_y, my_z)

        def rdma(src, dst, i):
            return pltpu.make_async_remote_copy(
                src_ref=src, dst_ref=dst,
                send_sem=send_sems.at[i], recv_sem=recv_sems.at[i],
                device_id=partner, device_id_type=pl.DeviceIdType.MESH)

        barrier_sem = pltpu.get_barrier_semaphore()
        pl.semaphore_signal(barrier_sem, inc=1, device_id=partner,
                            device_id_type=pl.DeviceIdType.MESH)

        x_mine = x_ref[...]
        xb = x_mine.astype(jnp.bfloat16)
        xb_ref[...] = xb
        w1c = jnp.concatenate(
            [w1_ref[j].astype(jnp.bfloat16) for j in range(e_per)], axis=1)
        w2c = jnp.concatenate(
            [w2_ref[j].astype(jnp.bfloat16) for j in range(e_per)], axis=0)

        pl.semaphore_wait(barrier_sem, 1)

        r_router = rdma(router_ref, ro_ref, 0)
        r_router.start()
        r_tok1 = rdma(xb_ref.at[pl.ds(0, half), :],
                      xo_ref.at[pl.ds(0, half), :], 1)
        r_tok1.start()

        r_router.wait()
        is0 = my_x == 0
        r_mine = router_ref[...]
        r_oth = ro_ref[...]
        r_full = jnp.where(is0,
                           jnp.concatenate([r_mine, r_oth], axis=1),
                           jnp.concatenate([r_oth, r_mine], axis=1))
        gm = jnp.dot(x_mine, r_full, preferred_element_type=jnp.float32)
        gm_ref[...] = gm
        r_gate = rdma(gm_ref, go_ref, 2)
        r_gate.start()
        r_tok2 = rdma(xb_ref.at[pl.ds(half, half), :],
                      xo_ref.at[pl.ds(half, half), :], 3)
        r_tok2.start()

        def expert_weights(gates):
            i1 = jnp.argmax(gates, axis=1)
            m1 = jnp.max(gates, axis=1)
            cols = lax.broadcasted_iota(jnp.int32, gates.shape, 1)
            masked = jnp.where(cols == i1[:, None], -jnp.inf, gates)
            i2 = jnp.argmax(masked, axis=1)
            m2 = jnp.max(masked, axis=1)
            e2 = jnp.exp(m2 - m1)
            w_top = 1.0 / (1.0 + e2)
            w_snd = e2 / (1.0 + e2)
            ws = []
            for j in range(e_per):
                e_glob = my_x * e_per + j
                w_e = (jnp.where(i1 == e_glob, w_top, 0.0)
                       + jnp.where(i2 == e_glob, w_snd, 0.0))
                ws.append(jnp.broadcast_to(
                    w_e[:, None].astype(jnp.bfloat16),
                    (gates.shape[0], f)))
            return jnp.concatenate(ws, axis=1)

        def block_out(tok_b, wcat):
            h = jnp.maximum(
                jnp.dot(tok_b, w1c, preferred_element_type=jnp.float32),
                0.0).astype(jnp.bfloat16)
            return jnp.dot(h * wcat, w2c, preferred_element_type=jnp.float32)

        w_mine = expert_weights(gm)
        acc_my1 = block_out(xb[:half], w_mine[:half])

        r_tok1.wait()
        r_gate.wait()
        w_for = expert_weights(go_ref[...])
        acc_f1 = block_out(xo_ref[pl.ds(0, half), :], w_for[:half])
        pb_ref[pl.ds(0, half), :] = acc_f1.astype(jnp.bfloat16)
        r_par1 = rdma(pb_ref.at[pl.ds(0, half), :],
                      po_ref.at[pl.ds(0, half), :], 4)
        r_par1.start()

        r_tok2.wait()
        acc_f2 = block_out(xo_ref[pl.ds(half, half), :], w_for[half:])
        pb_ref[pl.ds(half, half), :] = acc_f2.astype(jnp.bfloat16)
        r_par2 = rdma(pb_ref.at[pl.ds(half, half), :],
                      po_ref.at[pl.ds(half, half), :], 5)
        r_par2.start()

        acc_my2 = block_out(xb[half:], w_mine[half:])
        acc_my = jnp.concatenate([acc_my1, acc_my2], axis=0)

        r_par1.wait()
        r_par2.wait()
        out_ref[...] = acc_my + po_ref[...].astype(jnp.float32)

    return pl.pallas_call(
        body,
        out_shape=jax.ShapeDtypeStruct((t_per, d), jnp.float32),
        in_specs=[pl.BlockSpec(memory_space=pltpu.VMEM)] * 4,
        out_specs=pl.BlockSpec(memory_space=pltpu.VMEM),
        scratch_shapes=[
            pltpu.VMEM((d, e_per), jnp.float32),
            pltpu.VMEM((t_per, d), jnp.bfloat16),
            pltpu.VMEM((t_per, d), jnp.bfloat16),
            pltpu.VMEM((t_per, n_exp), jnp.float32),
            pltpu.VMEM((t_per, n_exp), jnp.float32),
            pltpu.VMEM((t_per, d), jnp.bfloat16),
            pltpu.VMEM((t_per, d), jnp.bfloat16),
            pltpu.SemaphoreType.DMA((6,)),
            pltpu.SemaphoreType.DMA((6,)),
        ],
        compiler_params=pltpu.CompilerParams(collective_id=0),
    )(x, router, W1, W2)


# device time: 24031 ns/iter; 1.2697x vs baseline; 1.0114x over previous
import jax
import jax.numpy as jnp
from jax import lax
from jax.experimental import pallas as pl
from jax.experimental.pallas import tpu as pltpu

X_SIZE = 2


def kernel(x, router, W1, W2):
    t_per, d = x.shape
    e_per, _, f = W1.shape
    n_exp = e_per * X_SIZE
    half = t_per // 2

    def body(x_ref, router_ref, w1_hbm, w2_hbm, out_ref,
             ro_ref, xb_ref, xo_ref, gm_ref, go_ref, pb_ref, po_ref,
             w1v_ref, w2v_ref, send_sems, recv_sems, copy_sems):
        my_x = lax.axis_index("x")
        my_y = lax.axis_index("y")
        my_z = lax.axis_index("z")
        partner = (1 - my_x, my_y, my_z)

        def rdma(src, dst, i):
            return pltpu.make_async_remote_copy(
                src_ref=src, dst_ref=dst,
                send_sem=send_sems.at[i], recv_sem=recv_sems.at[i],
                device_id=partner, device_id_type=pl.DeviceIdType.MESH)

        cp_w1 = pltpu.make_async_copy(w1_hbm, w1v_ref, copy_sems.at[0])
        cp_w1.start()
        cp_w2 = pltpu.make_async_copy(w2_hbm, w2v_ref, copy_sems.at[1])
        cp_w2.start()

        barrier_sem = pltpu.get_barrier_semaphore()
        pl.semaphore_signal(barrier_sem, inc=1, device_id=partner,
                            device_id_type=pl.DeviceIdType.MESH)

        x_mine = x_ref[...]
        xb = x_mine.astype(jnp.bfloat16)
        xb_ref[...] = xb

        pl.semaphore_wait(barrier_sem, 1)

        r_router = rdma(router_ref, ro_ref, 0)
        r_router.start()
        r_tok1 = rdma(xb_ref.at[pl.ds(0, half), :],
                      xo_ref.at[pl.ds(0, half), :], 1)
        r_tok1.start()

        r_router.wait()
        is0 = my_x == 0
        r_mine = router_ref[...]
        r_oth = ro_ref[...]
        r_full = jnp.where(is0,
                           jnp.concatenate([r_mine, r_oth], axis=1),
                           jnp.concatenate([r_oth, r_mine], axis=1))
        gm = jnp.dot(x_mine, r_full, preferred_element_type=jnp.float32)
        gm_ref[...] = gm
        r_gate = rdma(gm_ref, go_ref, 2)
        r_gate.start()
        r_tok2 = rdma(xb_ref.at[pl.ds(half, half), :],
                      xo_ref.at[pl.ds(half, half), :], 3)
        r_tok2.start()

        def expert_weights(gates):
            i1 = jnp.argmax(gates, axis=1)
            m1 = jnp.max(gates, axis=1)
            cols = lax.broadcasted_iota(jnp.int32, gates.shape, 1)
            masked = jnp.where(cols == i1[:, None], -jnp.inf, gates)
            i2 = jnp.argmax(masked, axis=1)
            m2 = jnp.max(masked, axis=1)
            e2 = jnp.exp(m2 - m1)
            w_top = 1.0 / (1.0 + e2)
            w_snd = e2 / (1.0 + e2)
            ws = []
            for j in range(e_per):
                e_glob = my_x * e_per + j
                w_e = (jnp.where(i1 == e_glob, w_top, 0.0)
                       + jnp.where(i2 == e_glob, w_snd, 0.0))
                ws.append(jnp.broadcast_to(
                    w_e[:, None].astype(jnp.bfloat16),
                    (gates.shape[0], f)))
            return jnp.concatenate(ws, axis=1)

        w_mine = expert_weights(gm)

        cp_w1.wait()
        w1c = jnp.concatenate(
            [w1v_ref[j].astype(jnp.bfloat16) for j in range(e_per)], axis=1)
        cp_w2.wait()
        w2c = jnp.concatenate(
            [w2v_ref[j].astype(jnp.bfloat16) for j in range(e_per)], axis=0)

        def block_out(tok_b, wcat):
            h = jnp.maximum(
                jnp.dot(tok_b, w1c, preferred_element_type=jnp.float32),
                0.0).astype(jnp.bfloat16)
            return jnp.dot(h * wcat, w2c, preferred_element_type=jnp.float32)

        acc_my1 = block_out(xb[:half], w_mine[:half])

        r_tok1.wait()
        r_gate.wait()
        w_for = expert_weights(go_ref[...])
        acc_f1 = block_out(xo_ref[pl.ds(0, half), :], w_for[:half])
        pb_ref[pl.ds(0, half), :] = acc_f1.astype(jnp.bfloat16)
        r_par1 = rdma(pb_ref.at[pl.ds(0, half), :],
                      po_ref.at[pl.ds(0, half), :], 4)
        r_par1.start()

        r_tok2.wait()
        acc_f2 = block_out(xo_ref[pl.ds(half, half), :], w_for[half:])
        pb_ref[pl.ds(half, half), :] = acc_f2.astype(jnp.bfloat16)
        r_par2 = rdma(pb_ref.at[pl.ds(half, half), :],
                      po_ref.at[pl.ds(half, half), :], 5)
        r_par2.start()

        acc_my2 = block_out(xb[half:], w_mine[half:])
        acc_my = jnp.concatenate([acc_my1, acc_my2], axis=0)

        r_par1.wait()
        r_par2.wait()
        out_ref[...] = acc_my + po_ref[...].astype(jnp.float32)

    return pl.pallas_call(
        body,
        out_shape=jax.ShapeDtypeStruct((t_per, d), jnp.float32),
        in_specs=[
            pl.BlockSpec(memory_space=pltpu.VMEM),
            pl.BlockSpec(memory_space=pltpu.VMEM),
            pl.BlockSpec(memory_space=pl.ANY),
            pl.BlockSpec(memory_space=pl.ANY),
        ],
        out_specs=pl.BlockSpec(memory_space=pltpu.VMEM),
        scratch_shapes=[
            pltpu.VMEM((d, e_per), jnp.float32),
            pltpu.VMEM((t_per, d), jnp.bfloat16),
            pltpu.VMEM((t_per, d), jnp.bfloat16),
            pltpu.VMEM((t_per, n_exp), jnp.float32),
            pltpu.VMEM((t_per, n_exp), jnp.float32),
            pltpu.VMEM((t_per, d), jnp.bfloat16),
            pltpu.VMEM((t_per, d), jnp.bfloat16),
            pltpu.VMEM((e_per, d, f), jnp.float32),
            pltpu.VMEM((e_per, f, d), jnp.float32),
            pltpu.SemaphoreType.DMA((6,)),
            pltpu.SemaphoreType.DMA((6,)),
            pltpu.SemaphoreType.DMA((2,)),
        ],
        compiler_params=pltpu.CompilerParams(collective_id=0),
    )(x, router, W1, W2)
